# Initial kernel scaffold; baseline (speedup 1.0000x reference)
import jax
import jax.numpy as jnp
from jax.experimental import pallas as pl


def kernel(p_2, p_3, p_4, p_5):
    raise NotImplementedError("write your pallas kernel here")


# speedup vs baseline: 23.0184x; 23.0184x over previous
import functools

import jax
import jax.numpy as jnp
import numpy as np
from jax import lax
from jax.experimental import pallas as pl
from jax.experimental.pallas import tpu as pltpu

_LEVELS = (
    ((4, 64, 64), 4.0),
    ((4, 32, 32), 8.0),
    ((4, 16, 16), 16.0),
    ((4, 8, 8), 32.0),
)
_ANCHOR_TAB = np.array([
    [[8, 8], [12, 10], [10, 14], [16, 16]],
    [[16, 16], [24, 20], [20, 28], [32, 32]],
    [[32, 32], [48, 40], [40, 56], [64, 64]],
    [[64, 64], [96, 80], [80, 112], [128, 128]],
], dtype=np.float32)

_N = sum(a * h * w for (a, h, w), _ in _LEVELS)
_ROWS = (_N + 127) // 128
_ROWS = ((_ROWS + 7) // 8) * 8
_NPAD = _ROWS * 128
_MAXP = 300
_TOPK = 2048
_IOU_T = 0.5
_SCORE_T = 0.05
_ONE_BITS = np.float32(1.0).view(np.int32).item()


def _build_consts() -> np.ndarray:
    cols = {k: [] for k in ("gx", "gy", "aw", "ah", "st", "vm")}
    for lvl, ((a, h, w), stride) in enumerate(_LEVELS):
        gy, gx = np.meshgrid(np.arange(h, dtype=np.float32),
                             np.arange(w, dtype=np.float32), indexing="ij")
        anc = _ANCHOR_TAB[lvl]
        cols["gx"].append(np.broadcast_to(gx[None], (a, h, w)).ravel())
        cols["gy"].append(np.broadcast_to(gy[None], (a, h, w)).ravel())
        cols["aw"].append(np.broadcast_to(anc[:, 0, None, None], (a, h, w)).ravel())
        cols["ah"].append(np.broadcast_to(anc[:, 1, None, None], (a, h, w)).ravel())
        cols["st"].append(np.full(a * h * w, stride, np.float32))
        cols["vm"].append(np.ones(a * h * w, np.float32))
    flat = [np.concatenate(cols[k]).astype(np.float32)
            for k in ("gx", "gy", "aw", "ah", "st", "vm")]
    stacked = np.stack([np.pad(f, (0, _NPAD - _N)) for f in flat])
    return stacked.reshape(6, _ROWS, 128)


_CONSTS = _build_consts()


def _nms_kernel(p_ref, c_ref, out_ref, x1_r, y1_r, x2_r, y2_r, ar_r, ob_r, cl_r):
    gx, gy = c_ref[0], c_ref[1]
    aw, ah = c_ref[2], c_ref[3]
    st, vm = c_ref[4], c_ref[5]

    x = (jax.nn.sigmoid(p_ref[0, 0]) + gx) * st
    y = (jax.nn.sigmoid(p_ref[0, 1]) + gy) * st
    w = jnp.exp(jnp.clip(p_ref[0, 2], -10.0, 8.0)) * aw
    h = jnp.exp(jnp.clip(p_ref[0, 3], -10.0, 8.0)) * ah
    obj = jax.nn.sigmoid(p_ref[0, 4])
    cls = jax.nn.sigmoid(p_ref[0, 5])

    x1 = x - w * 0.5
    y1 = y - h * 0.5
    x2 = x + w * 0.5
    y2 = y + h * 0.5
    x1_r[...] = x1
    y1_r[...] = y1
    x2_r[...] = x2
    y2_r[...] = y2
    ar_r[...] = jnp.maximum(x2 - x1, 0.0) * jnp.maximum(y2 - y1, 0.0)
    ob_r[...] = obj
    cl_r[...] = cls

    score = jnp.where(vm > 0.5, obj * cls, -1.0)
    sbits = lax.bitcast_convert_type(score, jnp.int32)
    idx = (lax.broadcasted_iota(jnp.int32, (_ROWS, 128), 0) * 128
           + lax.broadcasted_iota(jnp.int32, (_ROWS, 128), 1))

    def _bs_val(_, lohi):
        lo, hi = lohi
        mid = (lo + hi) // 2
        cnt = jnp.sum((sbits >= mid).astype(jnp.int32))
        big = cnt >= _TOPK
        return jnp.where(big, mid, lo), jnp.where(big, hi, mid)

    kbits, _ = lax.fori_loop(
        0, 31, _bs_val, (jnp.int32(0), jnp.int32(_ONE_BITS)))

    eq = sbits == kbits
    need = _TOPK - jnp.sum((sbits > kbits).astype(jnp.int32))

    def _bs_idx(_, lohi):
        lo, hi = lohi
        mid = (lo + hi) // 2
        cnt = jnp.sum((eq & (idx < mid)).astype(jnp.int32))
        ge = cnt >= need
        return jnp.where(ge, lo, mid), jnp.where(ge, mid, hi)

    _, cut = lax.fori_loop(
        0, 15, _bs_idx, (jnp.int32(0), jnp.int32(_NPAD)))

    member = (sbits > kbits) | (eq & (idx < cut))
    active0 = jnp.where(member, score, -1.0)
    lane = lax.broadcasted_iota(jnp.int32, (1, 128), 1)

    def _pick(k, active):
        m = jnp.max(active)
        pidx = jnp.min(jnp.where(active == m, idx, jnp.int32(_NPAD)))
        onehot = idx == pidx

        def ext(ref):
            return jnp.sum(jnp.where(onehot, ref[...], 0.0))

        x1p, y1p = ext(x1_r), ext(y1_r)
        x2p, y2p = ext(x2_r), ext(y2_r)
        obp, clp = ext(ob_r), ext(cl_r)
        areap = jnp.maximum(x2p - x1p, 0.0) * jnp.maximum(y2p - y1p, 0.0)

        ix1 = jnp.maximum(x1_r[...], x1p)
        iy1 = jnp.maximum(y1_r[...], y1p)
        ix2 = jnp.minimum(x2_r[...], x2p)
        iy2 = jnp.minimum(y2_r[...], y2p)
        inter = jnp.maximum(ix2 - ix1, 0.0) * jnp.maximum(iy2 - iy1, 0.0)
        iou = inter / (ar_r[...] + areap - inter + 1e-9)

        valid = jnp.where(m > _SCORE_T, 1.0, 0.0)
        row = jnp.zeros((1, 128), jnp.float32)
        for c, v in enumerate((x1p, y1p, x2p, y2p, obp, clp)):
            row = jnp.where(lane == c, v * valid, row)
        out_ref[0, pl.ds(k, 1), :] = row
        return jnp.where((iou > _IOU_T) | onehot, -1.0, active)

    lax.fori_loop(0, _MAXP, _pick, active0)


@jax.jit
def kernel(p_2, p_3, p_4, p_5):
    b = p_2.shape[0]
    flat = jnp.concatenate(
        [p.reshape(b, -1, 6) for p in (p_2, p_3, p_4, p_5)], axis=1)
    flat = jnp.pad(flat, ((0, 0), (0, _NPAD - _N), (0, 0)))
    flat = flat.transpose(0, 2, 1).reshape(b, 6, _ROWS, 128)
    consts = jnp.asarray(_CONSTS)

    out = pl.pallas_call(
        _nms_kernel,
        grid=(b,),
        in_specs=[
            pl.BlockSpec((1, 6, _ROWS, 128), lambda i: (i, 0, 0, 0)),
            pl.BlockSpec((6, _ROWS, 128), lambda i: (0, 0, 0)),
        ],
        out_specs=pl.BlockSpec((1, _MAXP, 128), lambda i: (i, 0, 0)),
        out_shape=jax.ShapeDtypeStruct((b, _MAXP, 128), jnp.float32),
        scratch_shapes=[pltpu.VMEM((_ROWS, 128), jnp.float32)] * 7,
    )(flat, consts)
    return out[:, :, :6]


# speedup vs baseline: 24.6328x; 1.0701x over previous
import functools

import jax
import jax.numpy as jnp
import numpy as np
from jax import lax
from jax.experimental import pallas as pl
from jax.experimental.pallas import tpu as pltpu

_LEVELS = (
    ((4, 64, 64), 4.0),
    ((4, 32, 32), 8.0),
    ((4, 16, 16), 16.0),
    ((4, 8, 8), 32.0),
)
_ANCHOR_TAB = np.array([
    [[8, 8], [12, 10], [10, 14], [16, 16]],
    [[16, 16], [24, 20], [20, 28], [32, 32]],
    [[32, 32], [48, 40], [40, 56], [64, 64]],
    [[64, 64], [96, 80], [80, 112], [128, 128]],
], dtype=np.float32)

_N = sum(a * h * w for (a, h, w), _ in _LEVELS)
_ROWS = (_N + 127) // 128
_ROWS = ((_ROWS + 7) // 8) * 8
_NPAD = _ROWS * 128
_MAXP = 300
_TOPK = 2048
_IOU_T = 0.5
_SCORE_T = 0.05
_ONE_BITS = np.float32(1.0).view(np.int32).item()


def _build_consts() -> np.ndarray:
    cols = {k: [] for k in ("gx", "gy", "aw", "ah", "st", "vm")}
    for lvl, ((a, h, w), stride) in enumerate(_LEVELS):
        gy, gx = np.meshgrid(np.arange(h, dtype=np.float32),
                             np.arange(w, dtype=np.float32), indexing="ij")
        anc = _ANCHOR_TAB[lvl]
        cols["gx"].append(np.broadcast_to(gx[None], (a, h, w)).ravel())
        cols["gy"].append(np.broadcast_to(gy[None], (a, h, w)).ravel())
        cols["aw"].append(np.broadcast_to(anc[:, 0, None, None], (a, h, w)).ravel())
        cols["ah"].append(np.broadcast_to(anc[:, 1, None, None], (a, h, w)).ravel())
        cols["st"].append(np.full(a * h * w, stride, np.float32))
        cols["vm"].append(np.ones(a * h * w, np.float32))
    flat = [np.concatenate(cols[k]).astype(np.float32)
            for k in ("gx", "gy", "aw", "ah", "st", "vm")]
    stacked = np.stack([np.pad(f, (0, _NPAD - _N)) for f in flat])
    return stacked.reshape(6, _ROWS, 128)


_CONSTS = _build_consts()


def _nms_kernel(p_ref, c_ref, out_ref, x1_r, y1_r, x2_r, y2_r, ar_r, ob_r, cl_r):
    gx, gy = c_ref[0], c_ref[1]
    aw, ah = c_ref[2], c_ref[3]
    st, vm = c_ref[4], c_ref[5]

    x = (jax.nn.sigmoid(p_ref[0, 0]) + gx) * st
    y = (jax.nn.sigmoid(p_ref[0, 1]) + gy) * st
    w = jnp.exp(jnp.clip(p_ref[0, 2], -10.0, 8.0)) * aw
    h = jnp.exp(jnp.clip(p_ref[0, 3], -10.0, 8.0)) * ah
    obj = jax.nn.sigmoid(p_ref[0, 4])
    cls = jax.nn.sigmoid(p_ref[0, 5])

    x1 = x - w * 0.5
    y1 = y - h * 0.5
    x2 = x + w * 0.5
    y2 = y + h * 0.5
    x1_r[...] = x1
    y1_r[...] = y1
    x2_r[...] = x2
    y2_r[...] = y2
    ar_r[...] = jnp.maximum(x2 - x1, 0.0) * jnp.maximum(y2 - y1, 0.0)
    ob_r[...] = obj
    cl_r[...] = cls

    score = jnp.where(vm > 0.5, obj * cls, -1.0)
    sbits = lax.bitcast_convert_type(score, jnp.int32)
    idx = (lax.broadcasted_iota(jnp.int32, (_ROWS, 128), 0) * 128
           + lax.broadcasted_iota(jnp.int32, (_ROWS, 128), 1))

    def _bs_val(_, lohi):
        lo, hi = lohi
        mid = (lo + hi) // 2
        cnt = jnp.sum((sbits >= mid).astype(jnp.int32))
        big = cnt >= _TOPK
        return jnp.where(big, mid, lo), jnp.where(big, hi, mid)

    kbits, _ = lax.fori_loop(
        0, 31, _bs_val, (jnp.int32(0), jnp.int32(_ONE_BITS)))

    eq = sbits == kbits
    need = _TOPK - jnp.sum((sbits > kbits).astype(jnp.int32))

    def _bs_idx(_, lohi):
        lo, hi = lohi
        mid = (lo + hi) // 2
        cnt = jnp.sum((eq & (idx < mid)).astype(jnp.int32))
        ge = cnt >= need
        return jnp.where(ge, lo, mid), jnp.where(ge, mid, hi)

    _, cut = lax.fori_loop(
        0, 15, _bs_idx, (jnp.int32(0), jnp.int32(_NPAD)))

    member = (sbits > kbits) | (eq & (idx < cut))
    active0 = jnp.where(member, score, -1.0)
    lane = lax.broadcasted_iota(jnp.int32, (1, 128), 1)

    def _pick(k, active):
        m = jnp.max(active)
        pidx = jnp.min(jnp.where(active == m, idx, jnp.int32(_NPAD)))
        pi = pidx // 128
        pj = pidx % 128

        def ext(ref):
            return jnp.sum(jnp.where(lane == pj, ref[pl.ds(pi, 1), :], 0.0))

        x1p, y1p = ext(x1_r), ext(y1_r)
        x2p, y2p = ext(x2_r), ext(y2_r)
        obp, clp = ext(ob_r), ext(cl_r)
        areap = jnp.maximum(x2p - x1p, 0.0) * jnp.maximum(y2p - y1p, 0.0)

        ix1 = jnp.maximum(x1_r[...], x1p)
        iy1 = jnp.maximum(y1_r[...], y1p)
        ix2 = jnp.minimum(x2_r[...], x2p)
        iy2 = jnp.minimum(y2_r[...], y2p)
        inter = jnp.maximum(ix2 - ix1, 0.0) * jnp.maximum(iy2 - iy1, 0.0)
        iou = inter / (ar_r[...] + areap - inter + 1e-9)

        valid = jnp.where(m > _SCORE_T, 1.0, 0.0)
        row = jnp.zeros((1, 128), jnp.float32)
        for c, v in enumerate((x1p, y1p, x2p, y2p, obp, clp)):
            row = jnp.where(lane == c, v * valid, row)
        out_ref[0, pl.ds(k, 1), :] = row
        return jnp.where(iou > _IOU_T, -1.0, active)

    lax.fori_loop(0, _MAXP, _pick, active0)


@jax.jit
def kernel(p_2, p_3, p_4, p_5):
    b = p_2.shape[0]
    flat = jnp.concatenate(
        [p.reshape(b, -1, 6) for p in (p_2, p_3, p_4, p_5)], axis=1)
    flat = jnp.pad(flat, ((0, 0), (0, _NPAD - _N), (0, 0)))
    flat = flat.transpose(0, 2, 1).reshape(b, 6, _ROWS, 128)
    consts = jnp.asarray(_CONSTS)

    out = pl.pallas_call(
        _nms_kernel,
        grid=(b,),
        in_specs=[
            pl.BlockSpec((1, 6, _ROWS, 128), lambda i: (i, 0, 0, 0)),
            pl.BlockSpec((6, _ROWS, 128), lambda i: (0, 0, 0)),
        ],
        out_specs=pl.BlockSpec((1, _MAXP, 128), lambda i: (i, 0, 0)),
        out_shape=jax.ShapeDtypeStruct((b, _MAXP, 128), jnp.float32),
        scratch_shapes=[pltpu.VMEM((_ROWS, 128), jnp.float32)] * 7,
    )(flat, consts)
    return out[:, :, :6]


# speedup vs baseline: 28.3370x; 1.1504x over previous
import jax
import jax.numpy as jnp
import numpy as np
from jax import lax
from jax.experimental import pallas as pl
from jax.experimental.pallas import tpu as pltpu

_LEVELS = (
    ((4, 64, 64), 4.0),
    ((4, 32, 32), 8.0),
    ((4, 16, 16), 16.0),
    ((4, 8, 8), 32.0),
)
_ANCHOR_TAB = np.array([
    [[8, 8], [12, 10], [10, 14], [16, 16]],
    [[16, 16], [24, 20], [20, 28], [32, 32]],
    [[32, 32], [48, 40], [40, 56], [64, 64]],
    [[64, 64], [96, 80], [80, 112], [128, 128]],
], dtype=np.float32)

_N = sum(a * h * w for (a, h, w), _ in _LEVELS)
_ROWS = (_N + 127) // 128
_ROWS = ((_ROWS + 7) // 8) * 8
_NPAD = _ROWS * 128
_B = 2
_MAXP = 300
_TOPK = 2048
_IOU_T = 0.5
_SCORE_T = 0.05
_ONE_BITS = np.float32(1.0).view(np.int32).item()


def _build_consts() -> np.ndarray:
    cols = {k: [] for k in ("gx", "gy", "aw", "ah", "st", "vm")}
    for lvl, ((a, h, w), stride) in enumerate(_LEVELS):
        gy, gx = np.meshgrid(np.arange(h, dtype=np.float32),
                             np.arange(w, dtype=np.float32), indexing="ij")
        anc = _ANCHOR_TAB[lvl]
        cols["gx"].append(np.broadcast_to(gx[None], (a, h, w)).ravel())
        cols["gy"].append(np.broadcast_to(gy[None], (a, h, w)).ravel())
        cols["aw"].append(np.broadcast_to(anc[:, 0, None, None], (a, h, w)).ravel())
        cols["ah"].append(np.broadcast_to(anc[:, 1, None, None], (a, h, w)).ravel())
        cols["st"].append(np.full(a * h * w, stride, np.float32))
        cols["vm"].append(np.ones(a * h * w, np.float32))
    flat = [np.concatenate(cols[k]).astype(np.float32)
            for k in ("gx", "gy", "aw", "ah", "st", "vm")]
    stacked = np.stack([np.pad(f, (0, _NPAD - _N)) for f in flat])
    return stacked.reshape(6, _ROWS, 128)


_CONSTS = _build_consts()
_X1, _Y1, _X2, _Y2, _AR, _OB, _CL = range(7)


def _nms_kernel(p_ref, c_ref, out_ref, box_r):
    gx, gy = c_ref[0], c_ref[1]
    aw, ah = c_ref[2], c_ref[3]
    st, vm = c_ref[4], c_ref[5]
    idx = (lax.broadcasted_iota(jnp.int32, (_ROWS, 128), 0) * 128
           + lax.broadcasted_iota(jnp.int32, (_ROWS, 128), 1))
    lane = lax.broadcasted_iota(jnp.int32, (1, 128), 1)

    sbits = []
    scores = []
    for b in range(_B):
        x = (jax.nn.sigmoid(p_ref[b, 0]) + gx) * st
        y = (jax.nn.sigmoid(p_ref[b, 1]) + gy) * st
        w = jnp.exp(jnp.clip(p_ref[b, 2], -10.0, 8.0)) * aw
        h = jnp.exp(jnp.clip(p_ref[b, 3], -10.0, 8.0)) * ah
        obj = jax.nn.sigmoid(p_ref[b, 4])
        cls = jax.nn.sigmoid(p_ref[b, 5])
        x1, y1 = x - w * 0.5, y - h * 0.5
        x2, y2 = x + w * 0.5, y + h * 0.5
        box_r[b, _X1], box_r[b, _Y1] = x1, y1
        box_r[b, _X2], box_r[b, _Y2] = x2, y2
        box_r[b, _AR] = jnp.maximum(x2 - x1, 0.0) * jnp.maximum(y2 - y1, 0.0)
        box_r[b, _OB], box_r[b, _CL] = obj, cls
        s = jnp.where(vm > 0.5, obj * cls, -1.0)
        scores.append(s)
        sbits.append(lax.bitcast_convert_type(s, jnp.int32))

    def _bs_val(_, state):
        out = []
        for b in range(_B):
            lo, hi = state[b]
            mid = (lo + hi) // 2
            big = jnp.sum((sbits[b] >= mid).astype(jnp.int32)) >= _TOPK
            out.append((jnp.where(big, mid, lo), jnp.where(big, hi, mid)))
        return tuple(out)

    kstate = lax.fori_loop(
        0, 31, _bs_val,
        tuple((jnp.int32(0), jnp.int32(_ONE_BITS)) for _ in range(_B)))
    kbits = [kstate[b][0] for b in range(_B)]

    eqs = [sbits[b] == kbits[b] for b in range(_B)]
    needs = [_TOPK - jnp.sum((sbits[b] > kbits[b]).astype(jnp.int32))
             for b in range(_B)]

    def _bs_idx(_, state):
        out = []
        for b in range(_B):
            lo, hi = state[b]
            mid = (lo + hi) // 2
            ge = jnp.sum((eqs[b] & (idx < mid)).astype(jnp.int32)) >= needs[b]
            out.append((jnp.where(ge, lo, mid), jnp.where(ge, mid, hi)))
        return tuple(out)

    cstate = lax.fori_loop(
        0, 15, _bs_idx,
        tuple((jnp.int32(0), jnp.int32(_NPAD)) for _ in range(_B)))

    actives = []
    for b in range(_B):
        member = (sbits[b] > kbits[b]) | (eqs[b] & (idx < cstate[b][1]))
        actives.append(jnp.where(member, scores[b], -1.0))

    def _pick(k, acts):
        nacts = []
        for b in range(_B):
            active = acts[b]
            m = jnp.max(active)
            pidx = jnp.min(jnp.where(active == m, idx, jnp.int32(_NPAD)))
            pi = pidx // 128
            pj = pidx % 128

            def ext(plane, pi=pi, pj=pj, b=b):
                return jnp.sum(jnp.where(
                    lane == pj, box_r[b, plane, pl.ds(pi, 1), :], 0.0))

            x1p, y1p, x2p, y2p = ext(_X1), ext(_Y1), ext(_X2), ext(_Y2)
            obp, clp = ext(_OB), ext(_CL)
            areap = (jnp.maximum(x2p - x1p, 0.0)
                     * jnp.maximum(y2p - y1p, 0.0))

            ix1 = jnp.maximum(box_r[b, _X1], x1p)
            iy1 = jnp.maximum(box_r[b, _Y1], y1p)
            ix2 = jnp.minimum(box_r[b, _X2], x2p)
            iy2 = jnp.minimum(box_r[b, _Y2], y2p)
            inter = jnp.maximum(ix2 - ix1, 0.0) * jnp.maximum(iy2 - iy1, 0.0)
            iou = inter / (box_r[b, _AR] + areap - inter + 1e-9)

            valid = jnp.where(m > _SCORE_T, 1.0, 0.0)
            row = jnp.zeros((1, 128), jnp.float32)
            for c, v in enumerate((x1p, y1p, x2p, y2p, obp, clp)):
                row = jnp.where(lane == c, v * valid, row)
            out_ref[b, pl.ds(k, 1), :] = row
            nacts.append(jnp.where(iou > _IOU_T, -1.0, active))
        return tuple(nacts)

    lax.fori_loop(0, _MAXP, _pick, tuple(actives))


@jax.jit
def kernel(p_2, p_3, p_4, p_5):
    flat = jnp.concatenate(
        [p.reshape(_B, -1, 6) for p in (p_2, p_3, p_4, p_5)], axis=1)
    flat = jnp.pad(flat, ((0, 0), (0, _NPAD - _N), (0, 0)))
    flat = flat.transpose(0, 2, 1).reshape(_B, 6, _ROWS, 128)
    consts = jnp.asarray(_CONSTS)

    out = pl.pallas_call(
        _nms_kernel,
        in_specs=[
            pl.BlockSpec((_B, 6, _ROWS, 128), lambda: (0, 0, 0, 0)),
            pl.BlockSpec((6, _ROWS, 128), lambda: (0, 0, 0)),
        ],
        out_specs=pl.BlockSpec((_B, _MAXP, 128), lambda: (0, 0, 0)),
        out_shape=jax.ShapeDtypeStruct((_B, _MAXP, 128), jnp.float32),
        scratch_shapes=[pltpu.VMEM((_B, 7, _ROWS, 128), jnp.float32)],
    )(flat, consts)
    return out[:, :, :6]


# speedup vs baseline: 49.4778x; 1.7460x over previous
import jax
import jax.numpy as jnp
import numpy as np
from jax import lax
from jax.experimental import pallas as pl
from jax.experimental.pallas import tpu as pltpu

_LEVELS = (
    ((4, 64, 64), 4.0),
    ((4, 32, 32), 8.0),
    ((4, 16, 16), 16.0),
    ((4, 8, 8), 32.0),
)
_ANCHOR_TAB = np.array([
    [[8, 8], [12, 10], [10, 14], [16, 16]],
    [[16, 16], [24, 20], [20, 28], [32, 32]],
    [[32, 32], [48, 40], [40, 56], [64, 64]],
    [[64, 64], [96, 80], [80, 112], [128, 128]],
], dtype=np.float32)

_N = sum(a * h * w for (a, h, w), _ in _LEVELS)
_ROWS = (_N + 127) // 128
_ROWS = ((_ROWS + 7) // 8) * 8
_NPAD = _ROWS * 128
_B = 2
_MAXP = 300
_TOPK = 2048
_IOU_T = 0.5
_SCORE_T = 0.05
_ONE_BITS = np.float32(1.0).view(np.int32).item()


def _build_consts() -> np.ndarray:
    cols = {k: [] for k in ("gx", "gy", "aw", "ah", "st", "vm")}
    for lvl, ((a, h, w), stride) in enumerate(_LEVELS):
        gy, gx = np.meshgrid(np.arange(h, dtype=np.float32),
                             np.arange(w, dtype=np.float32), indexing="ij")
        anc = _ANCHOR_TAB[lvl]
        cols["gx"].append(np.broadcast_to(gx[None], (a, h, w)).ravel())
        cols["gy"].append(np.broadcast_to(gy[None], (a, h, w)).ravel())
        cols["aw"].append(np.broadcast_to(anc[:, 0, None, None], (a, h, w)).ravel())
        cols["ah"].append(np.broadcast_to(anc[:, 1, None, None], (a, h, w)).ravel())
        cols["st"].append(np.full(a * h * w, stride, np.float32))
        cols["vm"].append(np.ones(a * h * w, np.float32))
    flat = [np.concatenate(cols[k]).astype(np.float32)
            for k in ("gx", "gy", "aw", "ah", "st", "vm")]
    stacked = np.stack([np.pad(f, (0, _NPAD - _N)) for f in flat])
    return stacked.reshape(6, _ROWS, 128)


_CONSTS = _build_consts()
_X1, _Y1, _X2, _Y2, _AR, _OB, _CL = range(7)


def _rmax(x):
    return jnp.max(jnp.max(x, axis=0, keepdims=True), axis=1, keepdims=True)


def _rmin(x):
    return jnp.min(jnp.min(x, axis=0, keepdims=True), axis=1, keepdims=True)


def _rsum(x):
    return jnp.sum(jnp.sum(x, axis=0, keepdims=True), axis=1, keepdims=True)


def _nms_kernel(p_ref, c_ref, out_ref, box_r):
    gx, gy = c_ref[0], c_ref[1]
    aw, ah = c_ref[2], c_ref[3]
    st, vm = c_ref[4], c_ref[5]
    idx = (lax.broadcasted_iota(jnp.int32, (_ROWS, 128), 0) * 128
           + lax.broadcasted_iota(jnp.int32, (_ROWS, 128), 1))
    lane = lax.broadcasted_iota(jnp.int32, (1, 128), 1)

    sbits = []
    scores = []
    for b in range(_B):
        x = (jax.nn.sigmoid(p_ref[b, 0]) + gx) * st
        y = (jax.nn.sigmoid(p_ref[b, 1]) + gy) * st
        w = jnp.exp(jnp.clip(p_ref[b, 2], -10.0, 8.0)) * aw
        h = jnp.exp(jnp.clip(p_ref[b, 3], -10.0, 8.0)) * ah
        obj = jax.nn.sigmoid(p_ref[b, 4])
        cls = jax.nn.sigmoid(p_ref[b, 5])
        x1, y1 = x - w * 0.5, y - h * 0.5
        x2, y2 = x + w * 0.5, y + h * 0.5
        box_r[b, _X1], box_r[b, _Y1] = x1, y1
        box_r[b, _X2], box_r[b, _Y2] = x2, y2
        box_r[b, _AR] = jnp.maximum(x2 - x1, 0.0) * jnp.maximum(y2 - y1, 0.0)
        box_r[b, _OB], box_r[b, _CL] = obj, cls
        s = jnp.where(vm > 0.5, obj * cls, -1.0)
        scores.append(s)
        sbits.append(lax.bitcast_convert_type(s, jnp.int32))

    def _bs_val(_, state):
        out = []
        for b in range(_B):
            lo, hi = state[b]
            mid = (lo + hi) // 2
            big = _rsum((sbits[b] >= mid).astype(jnp.int32)) >= _TOPK
            out.append((jnp.where(big, mid, lo), jnp.where(big, hi, mid)))
        return tuple(out)

    def _v11(val):
        return jnp.full((1, 1), val, jnp.int32)

    kstate = lax.fori_loop(
        0, 31, _bs_val,
        tuple((_v11(0), _v11(_ONE_BITS)) for _ in range(_B)))
    kbits = [kstate[b][0] for b in range(_B)]

    eqs = [sbits[b] == kbits[b] for b in range(_B)]
    needs = [_TOPK - _rsum((sbits[b] > kbits[b]).astype(jnp.int32))
             for b in range(_B)]

    def _bs_idx(_, state):
        out = []
        for b in range(_B):
            lo, hi = state[b]
            mid = (lo + hi) // 2
            ge = _rsum((eqs[b] & (idx < mid)).astype(jnp.int32)) >= needs[b]
            out.append((jnp.where(ge, lo, mid), jnp.where(ge, mid, hi)))
        return tuple(out)

    cstate = lax.fori_loop(
        0, 15, _bs_idx,
        tuple((_v11(0), _v11(_NPAD)) for _ in range(_B)))

    actives = []
    for b in range(_B):
        member = (sbits[b] > kbits[b]) | (eqs[b] & (idx < cstate[b][1]))
        actives.append(jnp.where(member, scores[b], -1.0))

    def _pick(k, acts):
        nacts = []
        for b in range(_B):
            active = acts[b]
            m = _rmax(active)
            pidx = _rmin(jnp.where(active == m, idx, jnp.int32(_NPAD)))
            onehot = idx == pidx

            x1v, y1v = box_r[b, _X1], box_r[b, _Y1]
            x2v, y2v = box_r[b, _X2], box_r[b, _Y2]
            obv, clv = box_r[b, _OB], box_r[b, _CL]

            def ext(plane, onehot=onehot):
                return _rmax(jnp.where(onehot, plane, -3.0e38))

            x1p, y1p, x2p, y2p = ext(x1v), ext(y1v), ext(x2v), ext(y2v)
            obp, clp = ext(obv), ext(clv)
            areap = (jnp.maximum(x2p - x1p, 0.0)
                     * jnp.maximum(y2p - y1p, 0.0))

            inter = (jnp.maximum(jnp.minimum(x2v, x2p)
                                 - jnp.maximum(x1v, x1p), 0.0)
                     * jnp.maximum(jnp.minimum(y2v, y2p)
                                   - jnp.maximum(y1v, y1p), 0.0))
            iou = inter / (box_r[b, _AR] + areap - inter + 1e-9)

            valid = jnp.where(m > _SCORE_T, 1.0, 0.0)
            row = jnp.zeros((1, 128), jnp.float32)
            for c, v in enumerate((x1p, y1p, x2p, y2p, obp, clp)):
                row = jnp.where(lane == c, v * valid, row)
            out_ref[b, pl.ds(k, 1), :] = row
            nacts.append(jnp.where(iou > _IOU_T, -1.0, active))
        return tuple(nacts)

    lax.fori_loop(0, _MAXP, _pick, tuple(actives))


@jax.jit
def kernel(p_2, p_3, p_4, p_5):
    flat = jnp.concatenate(
        [p.reshape(_B, -1, 6) for p in (p_2, p_3, p_4, p_5)], axis=1)
    flat = jnp.pad(flat, ((0, 0), (0, _NPAD - _N), (0, 0)))
    flat = flat.transpose(0, 2, 1).reshape(_B, 6, _ROWS, 128)
    consts = jnp.asarray(_CONSTS)

    out = pl.pallas_call(
        _nms_kernel,
        in_specs=[
            pl.BlockSpec((_B, 6, _ROWS, 128), lambda: (0, 0, 0, 0)),
            pl.BlockSpec((6, _ROWS, 128), lambda: (0, 0, 0)),
        ],
        out_specs=pl.BlockSpec((_B, _MAXP, 128), lambda: (0, 0, 0)),
        out_shape=jax.ShapeDtypeStruct((_B, _MAXP, 128), jnp.float32),
        scratch_shapes=[pltpu.VMEM((_B, 7, _ROWS, 128), jnp.float32)],
    )(flat, consts)
    return out[:, :, :6]


# speedup vs baseline: 62.1892x; 1.2569x over previous
import jax
import jax.numpy as jnp
import numpy as np
from jax import lax
from jax.experimental import pallas as pl
from jax.experimental.pallas import tpu as pltpu

_LEVELS = (
    ((4, 64, 64), 4.0),
    ((4, 32, 32), 8.0),
    ((4, 16, 16), 16.0),
    ((4, 8, 8), 32.0),
)
_ANCHOR_TAB = np.array([
    [[8, 8], [12, 10], [10, 14], [16, 16]],
    [[16, 16], [24, 20], [20, 28], [32, 32]],
    [[32, 32], [48, 40], [40, 56], [64, 64]],
    [[64, 64], [96, 80], [80, 112], [128, 128]],
], dtype=np.float32)

_N = sum(a * h * w for (a, h, w), _ in _LEVELS)
_ROWS = (_N + 127) // 128
_ROWS = ((_ROWS + 7) // 8) * 8
_NPAD = _ROWS * 128
_B = 2
_MAXP = 300
_TOPK = 2048
_KROWS = _TOPK // 128
_IOU_T = 0.5
_SCORE_T = 0.05
_ONE_BITS = np.float32(1.0).view(np.int32).item()


def _build_consts() -> np.ndarray:
    cols = {k: [] for k in ("gx", "gy", "aw", "ah", "st", "vm")}
    for lvl, ((a, h, w), stride) in enumerate(_LEVELS):
        gy, gx = np.meshgrid(np.arange(h, dtype=np.float32),
                             np.arange(w, dtype=np.float32), indexing="ij")
        anc = _ANCHOR_TAB[lvl]
        cols["gx"].append(np.broadcast_to(gx[None], (a, h, w)).ravel())
        cols["gy"].append(np.broadcast_to(gy[None], (a, h, w)).ravel())
        cols["aw"].append(np.broadcast_to(anc[:, 0, None, None], (a, h, w)).ravel())
        cols["ah"].append(np.broadcast_to(anc[:, 1, None, None], (a, h, w)).ravel())
        cols["st"].append(np.full(a * h * w, stride, np.float32))
        cols["vm"].append(np.ones(a * h * w, np.float32))
    flat = [np.concatenate(cols[k]).astype(np.float32)
            for k in ("gx", "gy", "aw", "ah", "st", "vm")]
    stacked = np.stack([np.pad(f, (0, _NPAD - _N)) for f in flat])
    return stacked.reshape(6, _ROWS, 128)


_CONSTS = _build_consts()


def _rmax(x):
    return jnp.max(jnp.max(x, axis=0, keepdims=True), axis=1, keepdims=True)


def _rmin(x):
    return jnp.min(jnp.min(x, axis=0, keepdims=True), axis=1, keepdims=True)


def _rsum(x):
    return jnp.sum(jnp.sum(x, axis=0, keepdims=True), axis=1, keepdims=True)


def _shift_flat(x, s, fill):
    rows = x.shape[0]
    if s % 128 == 0:
        r = s // 128
        pad = jnp.full((r, 128), fill, x.dtype)
        return jnp.concatenate([x[r:], pad], axis=0)
    down = jnp.concatenate([x[1:], jnp.full((1, 128), fill, x.dtype)], axis=0)
    return jnp.concatenate([x[:, s:], down[:, :s]], axis=1)


def _nms_kernel(p_ref, c_ref, out_ref):
    gx, gy = c_ref[0], c_ref[1]
    aw, ah = c_ref[2], c_ref[3]
    st, vm = c_ref[4], c_ref[5]
    idx = (lax.broadcasted_iota(jnp.int32, (_ROWS, 128), 0) * 128
           + lax.broadcasted_iota(jnp.int32, (_ROWS, 128), 1))
    lane = lax.broadcasted_iota(jnp.int32, (1, 128), 1)

    sbits = []
    boxes = []
    for b in range(_B):
        x = (jax.nn.sigmoid(p_ref[b, 0]) + gx) * st
        y = (jax.nn.sigmoid(p_ref[b, 1]) + gy) * st
        w = jnp.exp(jnp.clip(p_ref[b, 2], -10.0, 8.0)) * aw
        h = jnp.exp(jnp.clip(p_ref[b, 3], -10.0, 8.0)) * ah
        obj = jax.nn.sigmoid(p_ref[b, 4])
        cls = jax.nn.sigmoid(p_ref[b, 5])
        s = jnp.where(vm > 0.5, obj * cls, -1.0)
        boxes.append([x - w * 0.5, y - h * 0.5, x + w * 0.5, y + h * 0.5,
                      obj, cls, s])
        sbits.append(lax.bitcast_convert_type(s, jnp.int32))

    def _v11(val):
        return jnp.full((1, 1), val, jnp.int32)

    def _bs_val(_, state):
        out = []
        for b in range(_B):
            lo, hi = state[b]
            mid = (lo + hi) // 2
            big = _rsum((sbits[b] >= mid).astype(jnp.int32)) >= _TOPK
            out.append((jnp.where(big, mid, lo), jnp.where(big, hi, mid)))
        return tuple(out)

    kstate = lax.fori_loop(
        0, 31, _bs_val,
        tuple((_v11(0), _v11(_ONE_BITS)) for _ in range(_B)))
    kbits = [kstate[b][0] for b in range(_B)]

    eqs = [sbits[b] == kbits[b] for b in range(_B)]
    needs = [_TOPK - _rsum((sbits[b] > kbits[b]).astype(jnp.int32))
             for b in range(_B)]

    def _bs_idx(_, state):
        out = []
        for b in range(_B):
            lo, hi = state[b]
            mid = (lo + hi) // 2
            ge = _rsum((eqs[b] & (idx < mid)).astype(jnp.int32)) >= needs[b]
            out.append((jnp.where(ge, lo, mid), jnp.where(ge, mid, hi)))
        return tuple(out)

    cstate = lax.fori_loop(
        0, 15, _bs_idx,
        tuple((_v11(0), _v11(_NPAD)) for _ in range(_B)))

    comp = []
    for b in range(_B):
        member = ((sbits[b] > kbits[b])
                  | (eqs[b] & (idx < cstate[b][1]))).astype(jnp.int32)
        incl = member
        for k in range(7):
            s = 1 << k
            sh = jnp.concatenate(
                [jnp.zeros((_ROWS, s), jnp.int32), incl[:, :-s]], axis=1)
            incl = incl + sh
        rowtot = incl[:, 127:]
        ex = jnp.concatenate(
            [jnp.zeros((1, 1), jnp.int32), rowtot[:-1]], axis=0)
        for k in range(8):
            s = 1 << k
            sh = jnp.concatenate(
                [jnp.zeros((s, 1), jnp.int32), ex[:-s]], axis=0)
            ex = ex + sh
        rank = ex + (incl - member)
        dist = (idx - rank) * member
        planes = boxes[b] + [dist]
        for k in range(15):
            s = 1 << k
            bit = (planes[-1] >> k) & 1
            bit_in = _shift_flat(bit, s, 0)
            inc = bit_in == 1
            own = bit == 1
            newp = []
            for p in planes[:-1]:
                newp.append(jnp.where(inc, _shift_flat(p, s, 0.0), p))
            d = planes[-1]
            d_in = _shift_flat(d, s, 0) - s
            newd = jnp.where(inc, d_in, jnp.where(own, 0, d))
            planes = newp + [newd]
        comp.append([p[:_KROWS] for p in planes[:7]])

    idx16 = (lax.broadcasted_iota(jnp.int32, (_KROWS, 128), 0) * 128
             + lax.broadcasted_iota(jnp.int32, (_KROWS, 128), 1))

    cplanes = []
    actives = []
    for b in range(_B):
        x1c, y1c, x2c, y2c, obc, clc, sc = comp[b]
        arc = jnp.maximum(x2c - x1c, 0.0) * jnp.maximum(y2c - y1c, 0.0)
        cplanes.append((x1c, y1c, x2c, y2c, obc, clc, arc))
        actives.append(sc)

    def _pick(k, acts):
        nacts = []
        for b in range(_B):
            active = acts[b]
            x1v, y1v, x2v, y2v, obv, clv, arv = cplanes[b]
            m = _rmax(active)
            pidx = _rmin(jnp.where(active == m, idx16, jnp.int32(_TOPK)))
            onehot = idx16 == pidx

            def ext(plane, onehot=onehot):
                return _rmax(jnp.where(onehot, plane, -3.0e38))

            x1p, y1p, x2p, y2p = ext(x1v), ext(y1v), ext(x2v), ext(y2v)
            obp, clp = ext(obv), ext(clv)
            areap = (jnp.maximum(x2p - x1p, 0.0)
                     * jnp.maximum(y2p - y1p, 0.0))

            inter = (jnp.maximum(jnp.minimum(x2v, x2p)
                                 - jnp.maximum(x1v, x1p), 0.0)
                     * jnp.maximum(jnp.minimum(y2v, y2p)
                                   - jnp.maximum(y1v, y1p), 0.0))
            iou = inter / (arv + areap - inter + 1e-9)

            valid = jnp.where(m > _SCORE_T, 1.0, 0.0)
            row = jnp.zeros((1, 128), jnp.float32)
            for c, v in enumerate((x1p, y1p, x2p, y2p, obp, clp)):
                row = jnp.where(lane == c, v * valid, row)
            out_ref[b, pl.ds(k, 1), :] = row
            nacts.append(jnp.where(iou > _IOU_T, -1.0, active))
        return tuple(nacts)

    lax.fori_loop(0, _MAXP, _pick, tuple(actives))


@jax.jit
def kernel(p_2, p_3, p_4, p_5):
    flat = jnp.concatenate(
        [p.reshape(_B, -1, 6) for p in (p_2, p_3, p_4, p_5)], axis=1)
    flat = jnp.pad(flat, ((0, 0), (0, _NPAD - _N), (0, 0)))
    flat = flat.transpose(0, 2, 1).reshape(_B, 6, _ROWS, 128)
    consts = jnp.asarray(_CONSTS)

    out = pl.pallas_call(
        _nms_kernel,
        in_specs=[
            pl.BlockSpec((_B, 6, _ROWS, 128), lambda: (0, 0, 0, 0)),
            pl.BlockSpec((6, _ROWS, 128), lambda: (0, 0, 0)),
        ],
        out_specs=pl.BlockSpec((_B, _MAXP, 128), lambda: (0, 0, 0)),
        out_shape=jax.ShapeDtypeStruct((_B, _MAXP, 128), jnp.float32),
    )(flat, consts)
    return out[:, :, :6]


# speedup vs baseline: 62.5236x; 1.0054x over previous
import jax
import jax.numpy as jnp
import numpy as np
from jax import lax
from jax.experimental import pallas as pl
from jax.experimental.pallas import tpu as pltpu

_LEVELS = (
    ((4, 64, 64), 4.0),
    ((4, 32, 32), 8.0),
    ((4, 16, 16), 16.0),
    ((4, 8, 8), 32.0),
)
_ANCHOR_TAB = np.array([
    [[8, 8], [12, 10], [10, 14], [16, 16]],
    [[16, 16], [24, 20], [20, 28], [32, 32]],
    [[32, 32], [48, 40], [40, 56], [64, 64]],
    [[64, 64], [96, 80], [80, 112], [128, 128]],
], dtype=np.float32)

_N = sum(a * h * w for (a, h, w), _ in _LEVELS)
_ROWS = (_N + 127) // 128
_ROWS = ((_ROWS + 7) // 8) * 8
_NPAD = _ROWS * 128
_B = 2
_MAXP = 300
_TOPK = 2048
_KROWS = _TOPK // 128
_IOU_T = 0.5
_SCORE_T = 0.05
_ONE_BITS = np.float32(1.0).view(np.int32).item()


def _build_consts() -> np.ndarray:
    cols = {k: [] for k in ("gx", "gy", "aw", "ah", "st", "vm")}
    for lvl, ((a, h, w), stride) in enumerate(_LEVELS):
        gy, gx = np.meshgrid(np.arange(h, dtype=np.float32),
                             np.arange(w, dtype=np.float32), indexing="ij")
        anc = _ANCHOR_TAB[lvl]
        cols["gx"].append(np.broadcast_to(gx[None], (a, h, w)).ravel())
        cols["gy"].append(np.broadcast_to(gy[None], (a, h, w)).ravel())
        cols["aw"].append(np.broadcast_to(anc[:, 0, None, None], (a, h, w)).ravel())
        cols["ah"].append(np.broadcast_to(anc[:, 1, None, None], (a, h, w)).ravel())
        cols["st"].append(np.full(a * h * w, stride, np.float32))
        cols["vm"].append(np.ones(a * h * w, np.float32))
    flat = [np.concatenate(cols[k]).astype(np.float32)
            for k in ("gx", "gy", "aw", "ah", "st", "vm")]
    stacked = np.stack([np.pad(f, (0, _NPAD - _N)) for f in flat])
    return stacked.reshape(6, _ROWS, 128)


_CONSTS = _build_consts()


def _rmax(x):
    return jnp.max(jnp.max(x, axis=0, keepdims=True), axis=1, keepdims=True)


def _rmin(x):
    return jnp.min(jnp.min(x, axis=0, keepdims=True), axis=1, keepdims=True)


def _rsum(x):
    return jnp.sum(jnp.sum(x, axis=0, keepdims=True), axis=1, keepdims=True)


def _shift_flat(x, s, fill):
    rows = x.shape[0]
    if s % 128 == 0:
        r = s // 128
        pad = jnp.full((r, 128), fill, x.dtype)
        return jnp.concatenate([x[r:], pad], axis=0)
    down = jnp.concatenate([x[1:], jnp.full((1, 128), fill, x.dtype)], axis=0)
    return jnp.concatenate([x[:, s:], down[:, :s]], axis=1)


def _nms_kernel(p_ref, c_ref, out_ref):
    gx, gy = c_ref[0], c_ref[1]
    aw, ah = c_ref[2], c_ref[3]
    st, vm = c_ref[4], c_ref[5]
    idx = (lax.broadcasted_iota(jnp.int32, (_ROWS, 128), 0) * 128
           + lax.broadcasted_iota(jnp.int32, (_ROWS, 128), 1))
    lane = lax.broadcasted_iota(jnp.int32, (1, 128), 1)

    sbits = []
    boxes = []
    for b in range(_B):
        x = (jax.nn.sigmoid(p_ref[b, 0]) + gx) * st
        y = (jax.nn.sigmoid(p_ref[b, 1]) + gy) * st
        w = jnp.exp(jnp.clip(p_ref[b, 2], -10.0, 8.0)) * aw
        h = jnp.exp(jnp.clip(p_ref[b, 3], -10.0, 8.0)) * ah
        obj = jax.nn.sigmoid(p_ref[b, 4])
        cls = jax.nn.sigmoid(p_ref[b, 5])
        s = jnp.where(vm > 0.5, obj * cls, -1.0)
        boxes.append([x - w * 0.5, y - h * 0.5, x + w * 0.5, y + h * 0.5,
                      obj, cls])
        sbits.append(lax.bitcast_convert_type(s, jnp.int32))

    def _v11(val):
        return jnp.full((1, 1), val, jnp.int32)

    def _bs_val(_, state):
        out = []
        for b in range(_B):
            lo, hi = state[b]
            mid = (lo + hi) // 2
            big = _rsum((sbits[b] >= mid).astype(jnp.int32)) >= _TOPK
            out.append((jnp.where(big, mid, lo), jnp.where(big, hi, mid)))
        return tuple(out)

    kstate = lax.fori_loop(
        0, 31, _bs_val,
        tuple((_v11(0), _v11(_ONE_BITS)) for _ in range(_B)))
    kbits = [kstate[b][0] for b in range(_B)]

    eqs = [sbits[b] == kbits[b] for b in range(_B)]
    needs = [_TOPK - _rsum((sbits[b] > kbits[b]).astype(jnp.int32))
             for b in range(_B)]

    def _bs_idx(_, state):
        out = []
        for b in range(_B):
            lo, hi = state[b]
            mid = (lo + hi) // 2
            ge = _rsum((eqs[b] & (idx < mid)).astype(jnp.int32)) >= needs[b]
            out.append((jnp.where(ge, lo, mid), jnp.where(ge, mid, hi)))
        return tuple(out)

    cstate = lax.fori_loop(
        0, 15, _bs_idx,
        tuple((_v11(0), _v11(_NPAD)) for _ in range(_B)))

    comp = []
    for b in range(_B):
        member = ((sbits[b] > kbits[b])
                  | (eqs[b] & (idx < cstate[b][1]))).astype(jnp.int32)
        incl = member
        for k in range(7):
            s = 1 << k
            sh = jnp.concatenate(
                [jnp.zeros((_ROWS, s), jnp.int32), incl[:, :-s]], axis=1)
            incl = incl + sh
        rowtot = incl[:, 127:]
        ex = jnp.concatenate(
            [jnp.zeros((1, 1), jnp.int32), rowtot[:-1]], axis=0)
        for k in range(8):
            s = 1 << k
            sh = jnp.concatenate(
                [jnp.zeros((s, 1), jnp.int32), ex[:-s]], axis=0)
            ex = ex + sh
        rank = ex + (incl - member)
        dist = (idx - rank) * member
        planes = boxes[b] + [dist]
        for k in range(15):
            s = 1 << k
            bit = (planes[-1] >> k) & 1
            bit_in = _shift_flat(bit, s, 0)
            inc = bit_in == 1
            own = bit == 1
            newp = []
            for p in planes[:-1]:
                newp.append(jnp.where(inc, _shift_flat(p, s, 0.0), p))
            d = planes[-1]
            d_in = _shift_flat(d, s, 0) - s
            newd = jnp.where(inc, d_in, jnp.where(own, 0, d))
            planes = newp + [newd]
        comp.append([p[:_KROWS] for p in planes[:6]])

    idx16 = (lax.broadcasted_iota(jnp.int32, (_KROWS, 128), 0) * 128
             + lax.broadcasted_iota(jnp.int32, (_KROWS, 128), 1))

    cplanes = []
    actives = []
    for b in range(_B):
        x1c, y1c, x2c, y2c, obc, clc = comp[b]
        arc = jnp.maximum(x2c - x1c, 0.0) * jnp.maximum(y2c - y1c, 0.0)
        cplanes.append((x1c, y1c, x2c, y2c, obc, clc, arc))
        actives.append(obc * clc)

    def _pick(k, acts):
        nacts = []
        for b in range(_B):
            active = acts[b]
            x1v, y1v, x2v, y2v, obv, clv, arv = cplanes[b]
            m = _rmax(active)
            pidx = _rmin(jnp.where(active == m, idx16, jnp.int32(_TOPK)))
            onehot = idx16 == pidx

            def ext(plane, onehot=onehot):
                return _rmax(jnp.where(onehot, plane, -3.0e38))

            x1p, y1p, x2p, y2p = ext(x1v), ext(y1v), ext(x2v), ext(y2v)
            obp, clp = ext(obv), ext(clv)
            areap = (jnp.maximum(x2p - x1p, 0.0)
                     * jnp.maximum(y2p - y1p, 0.0))

            inter = (jnp.maximum(jnp.minimum(x2v, x2p)
                                 - jnp.maximum(x1v, x1p), 0.0)
                     * jnp.maximum(jnp.minimum(y2v, y2p)
                                   - jnp.maximum(y1v, y1p), 0.0))
            iou = inter / (arv + areap - inter + 1e-9)

            valid = jnp.where(m > _SCORE_T, 1.0, 0.0)
            row = jnp.zeros((1, 128), jnp.float32)
            for c, v in enumerate((x1p, y1p, x2p, y2p, obp, clp)):
                row = jnp.where(lane == c, v * valid, row)
            out_ref[b, pl.ds(k, 1), :] = row
            nacts.append(jnp.where(iou > _IOU_T, -1.0, active))
        return tuple(nacts)

    lax.fori_loop(0, _MAXP, _pick, tuple(actives))


@jax.jit
def kernel(p_2, p_3, p_4, p_5):
    flat = jnp.concatenate(
        [p.reshape(_B, -1, 6) for p in (p_2, p_3, p_4, p_5)], axis=1)
    flat = jnp.pad(flat, ((0, 0), (0, _NPAD - _N), (0, 0)))
    flat = flat.transpose(0, 2, 1).reshape(_B, 6, _ROWS, 128)
    consts = jnp.asarray(_CONSTS)

    out = pl.pallas_call(
        _nms_kernel,
        in_specs=[
            pl.BlockSpec((_B, 6, _ROWS, 128), lambda: (0, 0, 0, 0)),
            pl.BlockSpec((6, _ROWS, 128), lambda: (0, 0, 0)),
        ],
        out_specs=pl.BlockSpec((_B, _MAXP, 128), lambda: (0, 0, 0)),
        out_shape=jax.ShapeDtypeStruct((_B, _MAXP, 128), jnp.float32),
    )(flat, consts)
    return out[:, :, :6]
